# Initial kernel scaffold; baseline (speedup 1.0000x reference)
import jax
import jax.numpy as jnp
from jax.experimental import pallas as pl


def kernel(token_ids, embeddings):
    raise NotImplementedError("write your pallas kernel here")


# speedup vs baseline: 1.0425x; 1.0425x over previous
import jax
import jax.numpy as jnp
from jax.experimental import pallas as pl
from jax.experimental.pallas import tpu as pltpu
from jax.experimental.pallas import tpu_sc as plsc

_WINDOW = 128


def kernel(token_ids, embeddings):
    batch, seq = token_ids.shape
    num_idx = batch * seq
    dim = embeddings.shape[1]
    flat_ids = token_ids.reshape(1, num_idx)

    mesh = plsc.VectorSubcoreMesh(core_axis_name="core",
                                  subcore_axis_name="subcore")

    @pl.kernel(
        out_type=jax.ShapeDtypeStruct((num_idx, dim), embeddings.dtype),
        mesh=mesh,
        compiler_params=pltpu.CompilerParams(use_tc_tiling_on_sc=False),
    )
    def sc_gather(table_hbm, ids_hbm, out_hbm):
        def body(ids_vmem, out_vmem):
            pltpu.sync_copy(table_hbm.at[ids_vmem.at[0]], out_vmem)

        pltpu.emit_pipeline(
            body,
            grid=(num_idx // _WINDOW,),
            in_specs=[pl.BlockSpec((1, _WINDOW), index_map=lambda i: (0, i))],
            out_specs=[pl.BlockSpec((_WINDOW, dim), index_map=lambda i: (i, 0))],
            core_axis_name=("core", "subcore"),
            dimension_semantics=(pltpu.PARALLEL,),
        )(ids_hbm, out_hbm)

    return sc_gather(embeddings, flat_ids).reshape(batch, seq, dim)


# speedup vs baseline: 1.0465x; 1.0039x over previous
import jax
import jax.numpy as jnp
from jax.experimental import pallas as pl
from jax.experimental.pallas import tpu as pltpu
from jax.experimental.pallas import tpu_sc as plsc

_ROWS = 8
_W = 128
_WINDOW = _ROWS * _W


def kernel(token_ids, embeddings):
    batch, seq = token_ids.shape
    num_idx = batch * seq
    dim = embeddings.shape[1]
    ids2d = token_ids.reshape(num_idx // _W, _W)

    mesh = plsc.VectorSubcoreMesh(core_axis_name="core",
                                  subcore_axis_name="subcore")

    @pl.kernel(
        out_type=jax.ShapeDtypeStruct((num_idx, dim), embeddings.dtype),
        mesh=mesh,
        compiler_params=pltpu.CompilerParams(use_tc_tiling_on_sc=False),
    )
    def sc_gather(table_hbm, ids_hbm, out_hbm):
        def body(ids_vmem, out_vmem):
            for j in range(_ROWS):
                pltpu.sync_copy(table_hbm.at[ids_vmem.at[j]],
                                out_vmem.at[pl.ds(j * _W, _W)])

        pltpu.emit_pipeline(
            body,
            grid=(num_idx // _WINDOW,),
            in_specs=[pl.BlockSpec((_ROWS, _W), index_map=lambda i: (i, 0))],
            out_specs=[pl.BlockSpec((_WINDOW, dim), index_map=lambda i: (i, 0))],
            core_axis_name=("core", "subcore"),
            dimension_semantics=(pltpu.PARALLEL,),
        )(ids_hbm, out_hbm)

    return sc_gather(embeddings, ids2d).reshape(batch, seq, dim)


# speedup vs baseline: 1.4289x; 1.3654x over previous
import jax
import jax.numpy as jnp
from jax.experimental import pallas as pl
from jax.experimental.pallas import tpu as pltpu
from jax.experimental.pallas import tpu_sc as plsc

_ROWS = 8


def kernel(token_ids, embeddings):
    batch, seq = token_ids.shape
    dim = embeddings.shape[1]

    mesh = plsc.VectorSubcoreMesh(core_axis_name="core",
                                  subcore_axis_name="subcore")

    @pl.kernel(
        out_type=jax.ShapeDtypeStruct((batch, seq, dim), embeddings.dtype),
        mesh=mesh,
        compiler_params=pltpu.CompilerParams(use_tc_tiling_on_sc=False),
    )
    def sc_gather(table_hbm, ids_hbm, out_hbm):
        def body(ids_vmem, out_vmem):
            for j in range(_ROWS):
                pltpu.sync_copy(table_hbm.at[ids_vmem.at[j]],
                                out_vmem.at[j])

        pltpu.emit_pipeline(
            body,
            grid=(batch // _ROWS,),
            in_specs=[pl.BlockSpec((_ROWS, seq), index_map=lambda i: (i, 0))],
            out_specs=[pl.BlockSpec((_ROWS, seq, dim),
                                    index_map=lambda i: (i, 0, 0))],
            core_axis_name=("core", "subcore"),
            dimension_semantics=(pltpu.PARALLEL,),
        )(ids_hbm, out_hbm)

    return sc_gather(embeddings, token_ids)
